# baseline (speedup 1.0000x reference)
import functools

import jax
import jax.numpy as jnp
from jax import lax
from jax.experimental import pallas as pl
from jax.experimental.pallas import tpu as pltpu
from jax.experimental.pallas import tpu_sc as plsc

N_NODES = 10000
N_EDGES = 320000
F = 128
HIDDEN = 128
NUM_CLASSES = 64
COEF = 7
SPLINE_ORDER = 3
GRID_SIZE = 4

NC = 2
NS = 16
NW = NC * NS
CH = 80
NQ = 32
NCH = 4 * NQ
EPW = NCH * CH
E_PAD = NW * EPW
ACC_ROWS = 10016
DUMMY_ROW = N_NODES
ROWS_PER_SUB = 624
ROWS_TAIL_OFF = NS * ROWS_PER_SUB
ROWS_TAIL = N_NODES - ROWS_TAIL_OFF
ZERO_TAIL = ACC_ROWS - ROWS_TAIL_OFF


def _sc_segsum(vals, cidx, zeros):
    mesh = plsc.VectorSubcoreMesh(core_axis_name="c", subcore_axis_name="s")

    @functools.partial(
        pl.kernel,
        out_type=jax.ShapeDtypeStruct((NC, N_NODES, F), jnp.float32),
        mesh=mesh,
        scratch_types=[
            pltpu.VMEM((8, CH), jnp.int32),
            pltpu.VMEM((8, CH), jnp.int32),
            pltpu.VMEM((CH, F), jnp.float32),
            pltpu.VMEM((CH, F), jnp.float32),
            pltpu.VMEM((CH, F), jnp.float32),
            pltpu.VMEM((CH, F), jnp.float32),
            pltpu.VMEM_SHARED((ACC_ROWS, F), jnp.float32),
            pltpu.SemaphoreType.DMA,
            pltpu.SemaphoreType.DMA,
            pltpu.SemaphoreType.DMA,
            pltpu.SemaphoreType.DMA,
            pltpu.SemaphoreType.DMA,
            pltpu.SemaphoreType.DMA,
            pltpu.SemaphoreType.DMA,
            pltpu.SemaphoreType.DMA,
            pltpu.SemaphoreType.DMA,
            pltpu.SemaphoreType.DMA,
        ],
    )
    def seg_kernel(vals_hbm, cidx_hbm, zeros_hbm, out_hbm,
                   idx0, idx1, buf0, buf1, buf2, buf3, acc,
                   g0, g1, g2, g3, s0, s1, s2, s3, i0sem, i1sem):
        cid = lax.axis_index("c")
        sid = lax.axis_index("s")
        wid = sid * NC + cid
        pltpu.sync_copy(zeros_hbm.at[pl.ds(sid * ROWS_PER_SUB, ROWS_PER_SUB)],
                        acc.at[pl.ds(sid * ROWS_PER_SUB, ROWS_PER_SUB)])

        @pl.when(sid == NS - 1)
        def _zero_tail():
            pltpu.sync_copy(zeros_hbm.at[pl.ds(ROWS_TAIL_OFF, ZERO_TAIL)],
                            acc.at[pl.ds(ROWS_TAIL_OFF, ZERO_TAIL)])

        plsc.subcore_barrier()

        bufs = (buf0, buf1, buf2, buf3)
        gsem = (g0, g1, g2, g3)
        ssem = (s0, s1, s2, s3)

        def fetch_idx(q, ibuf, sem):
            return pltpu.async_copy(cidx_hbm.at[wid, q], ibuf, sem)

        def gather(ibuf, k, sem):
            return pltpu.async_copy(vals_hbm.at[ibuf.at[2 * k]], bufs[k], sem)

        def scat(ibuf, k, sem):
            return pltpu.async_copy(bufs[k], acc.at[ibuf.at[2 * k + 1]], sem,
                                    add=True)

        def drain_g(k):
            pltpu.make_async_copy(vals_hbm.at[pl.ds(0, CH)], bufs[k],
                                  gsem[k]).wait()

        def drain_s(k):
            pltpu.make_async_copy(vals_hbm.at[pl.ds(0, CH)], bufs[k],
                                  ssem[k]).wait()

        def drain_i(ibuf, sem):
            pltpu.make_async_copy(cidx_hbm.at[0, 0], ibuf, sem).wait()

        fetch_idx(0, idx0, i0sem)
        drain_i(idx0, i0sem)
        fetch_idx(1, idx1, i1sem)
        for k in range(4):
            gather(idx0, k, gsem[k])

        def two_quads(m, carry):
            a = 2 * m
            for k in range(4):
                drain_g(k)
                scat(idx0, k, ssem[k])
            drain_i(idx1, i1sem)
            for k in range(4):
                drain_s(k)
                gather(idx1, k, gsem[k])
            fetch_idx(a + 2, idx0, i0sem)
            for k in range(4):
                drain_g(k)
                scat(idx1, k, ssem[k])
            drain_i(idx0, i0sem)
            for k in range(4):
                drain_s(k)
                gather(idx0, k, gsem[k])
            fetch_idx(a + 3, idx1, i1sem)
            return carry

        lax.fori_loop(0, NQ // 2 - 1, two_quads, 0)
        for k in range(4):
            drain_g(k)
            scat(idx0, k, ssem[k])
        drain_i(idx1, i1sem)
        for k in range(4):
            drain_s(k)
            gather(idx1, k, gsem[k])
        for k in range(4):
            drain_g(k)
            scat(idx1, k, ssem[k])
        for k in range(4):
            drain_s(k)

        plsc.subcore_barrier()
        pltpu.sync_copy(acc.at[pl.ds(sid * ROWS_PER_SUB, ROWS_PER_SUB)],
                        out_hbm.at[cid, pl.ds(sid * ROWS_PER_SUB, ROWS_PER_SUB)])

        @pl.when(sid == NS - 1)
        def _out_tail():
            pltpu.sync_copy(acc.at[pl.ds(ROWS_TAIL_OFF, ROWS_TAIL)],
                            out_hbm.at[cid, pl.ds(ROWS_TAIL_OFF, ROWS_TAIL)])

    return seg_kernel(vals, cidx, zeros)


BT = 400


def _bspline_bases(z):
    t = [0.5 * i - 2.5 for i in range(GRID_SIZE + 2 * SPLINE_ORDER + 1)]
    b = [jnp.where((z >= t[i]) & (z < t[i + 1]), 1.0, 0.0).astype(z.dtype)
         for i in range(len(t) - 1)]
    for j in range(1, SPLINE_ORDER + 1):
        b = [(z - t[i]) / (t[i + j] - t[i]) * b[i]
             + (t[i + j + 1] - z) / (t[i + j + 1] - t[i + 1]) * b[i + 1]
             for i in range(len(b) - 1)]
    return b


def _silu(z):
    return z * (1.0 / (1.0 + jnp.exp(-z)))


def _kan1_body(x_ref, p_ref, bwt_ref, sw_ref, z_ref, h_ref):
    z = x_ref[...] + p_ref[0] + p_ref[1]
    z_ref[...] = z
    acc = jnp.dot(_silu(z), bwt_ref[...], preferred_element_type=jnp.float32)
    for c, bc in enumerate(_bspline_bases(z)):
        acc += jnp.dot(bc, sw_ref[c], preferred_element_type=jnp.float32)
    h_ref[...] = acc


def _kan1(x, p, bwt, sw):
    grid = (N_NODES // BT,)
    return pl.pallas_call(
        _kan1_body,
        grid=grid,
        in_specs=[
            pl.BlockSpec((BT, F), lambda i: (i, 0)),
            pl.BlockSpec((NC, BT, F), lambda i: (0, i, 0)),
            pl.BlockSpec((F, HIDDEN), lambda i: (0, 0)),
            pl.BlockSpec((COEF, F, HIDDEN), lambda i: (0, 0, 0)),
        ],
        out_specs=[
            pl.BlockSpec((BT, F), lambda i: (i, 0)),
            pl.BlockSpec((BT, HIDDEN), lambda i: (i, 0)),
        ],
        out_shape=[
            jax.ShapeDtypeStruct((N_NODES, F), jnp.float32),
            jax.ShapeDtypeStruct((N_NODES, HIDDEN), jnp.float32),
        ],
    )(x, p, bwt, sw)


def _kan2_body(z1_ref, h1_ref, q_ref, bwta_ref, bwtb_ref, swa_ref, swb_ref,
               o_ref):
    z1 = z1_ref[...]
    h2 = h1_ref[...] + q_ref[0] + q_ref[1]
    acc = jnp.dot(_silu(z1), bwta_ref[...], preferred_element_type=jnp.float32)
    acc += jnp.dot(_silu(h2), bwtb_ref[...], preferred_element_type=jnp.float32)
    for c, bc in enumerate(_bspline_bases(z1)):
        acc += jnp.dot(bc, swa_ref[c], preferred_element_type=jnp.float32)
    for c, bc in enumerate(_bspline_bases(h2)):
        acc += jnp.dot(bc, swb_ref[c], preferred_element_type=jnp.float32)
    o_ref[...] = acc


def _kan2(z1, h1, q, bwta, bwtb, swa, swb):
    grid = (N_NODES // BT,)
    return pl.pallas_call(
        _kan2_body,
        grid=grid,
        in_specs=[
            pl.BlockSpec((BT, F), lambda i: (i, 0)),
            pl.BlockSpec((BT, HIDDEN), lambda i: (i, 0)),
            pl.BlockSpec((NC, BT, HIDDEN), lambda i: (0, i, 0)),
            pl.BlockSpec((F, NUM_CLASSES), lambda i: (0, 0)),
            pl.BlockSpec((HIDDEN, NUM_CLASSES), lambda i: (0, 0)),
            pl.BlockSpec((COEF, F, NUM_CLASSES), lambda i: (0, 0, 0)),
            pl.BlockSpec((COEF, HIDDEN, NUM_CLASSES), lambda i: (0, 0, 0)),
        ],
        out_specs=pl.BlockSpec((BT, NUM_CLASSES), lambda i: (i, 0)),
        out_shape=jax.ShapeDtypeStruct((N_NODES, NUM_CLASSES), jnp.float32),
    )(z1, h1, q, bwta, bwtb, swa, swb)


def kernel(x, edge_index, base_w1, spline_w1, scaler1,
           base_w2, spline_w2, scaler2):
    pad = E_PAD - N_EDGES
    src = jnp.concatenate([edge_index[0], jnp.zeros((pad,), jnp.int32)])
    dst = jnp.concatenate(
        [edge_index[1], jnp.full((pad,), DUMMY_ROW, jnp.int32)])
    cidx = jnp.stack(
        [src.reshape(NW, NQ, 4, CH), dst.reshape(NW, NQ, 4, CH)], axis=3
    ).reshape(NW, NQ, 8, CH)
    zeros = jnp.zeros((ACC_ROWS, F), jnp.float32)

    bwt1 = base_w1.T
    sw1 = (spline_w1 * scaler1[:, :, None]).transpose(2, 1, 0)
    bwt2a = base_w2[:, :F].T
    bwt2b = base_w2[:, F:].T
    sw2 = (spline_w2 * scaler2[:, :, None]).transpose(2, 1, 0)
    sw2a = sw2[:, :F, :]
    sw2b = sw2[:, F:, :]

    p = _sc_segsum(x, cidx, zeros)
    z1, h1 = _kan1(x, p, bwt1, sw1)
    q = _sc_segsum(h1, cidx, zeros)
    return _kan2(z1, h1, q, bwt2a, bwt2b, sw2a, sw2b)


# speedup vs baseline: 3.0523x; 3.0523x over previous
import functools

import jax
import jax.numpy as jnp
from jax import lax
from jax.experimental import pallas as pl
from jax.experimental.pallas import tpu as pltpu
from jax.experimental.pallas import tpu_sc as plsc

N_NODES = 10000
N_EDGES = 320000
F = 128
HIDDEN = 128
NUM_CLASSES = 64
COEF = 7
SPLINE_ORDER = 3
GRID_SIZE = 4

NC = 2
NS = 16
NW = NC * NS
EPW = N_EDGES // NW
CH = 80
NCH = EPW // CH
ROWS_PER_SUB = 624
ROWS_TAIL_OFF = NS * ROWS_PER_SUB
ROWS_TAIL = N_NODES - ROWS_TAIL_OFF


def _sc_segsum(vals, src, dst3, zeros):
    mesh = plsc.VectorSubcoreMesh(core_axis_name="c", subcore_axis_name="s")

    @functools.partial(
        pl.kernel,
        out_type=jax.ShapeDtypeStruct((NC, N_NODES, F), jnp.float32),
        mesh=mesh,
        scratch_types=[
            pltpu.VMEM((EPW,), jnp.int32),
            pltpu.VMEM((NCH, CH), jnp.int32),
            pltpu.VMEM((CH, F), jnp.float32),
            pltpu.VMEM((CH, F), jnp.float32),
            pltpu.VMEM_SHARED((N_NODES, F), jnp.float32),
            pltpu.SemaphoreType.DMA,
            pltpu.SemaphoreType.DMA,
            pltpu.SemaphoreType.DMA,
        ],
    )
    def seg_kernel(vals_hbm, src_hbm, dst_hbm, zeros_hbm, out_hbm,
                   srcv, dstv, bufa, bufb, acc, sema, semb, semi):
        cid = lax.axis_index("c")
        sid = lax.axis_index("s")
        wid = sid * NC + cid
        base = wid * EPW
        cp_src = pltpu.async_copy(src_hbm.at[pl.ds(base, EPW)], srcv, semi)
        pltpu.sync_copy(dst_hbm.at[wid], dstv)
        pltpu.sync_copy(zeros_hbm.at[pl.ds(sid * ROWS_PER_SUB, ROWS_PER_SUB)],
                        acc.at[pl.ds(sid * ROWS_PER_SUB, ROWS_PER_SUB)])

        @pl.when(sid == NS - 1)
        def _zero_tail():
            pltpu.sync_copy(zeros_hbm.at[pl.ds(ROWS_TAIL_OFF, ROWS_TAIL)],
                            acc.at[pl.ds(ROWS_TAIL_OFF, ROWS_TAIL)])

        cp_src.wait()
        plsc.subcore_barrier()

        def gather(c, buf, sem):
            return pltpu.async_copy(vals_hbm.at[srcv.at[pl.ds(c * CH, CH)]],
                                    buf, sem)

        gather(0, bufa, sema)

        def pair(p, carry):
            c0 = 2 * p
            gather(c0 + 1, bufb, semb)
            pltpu.make_async_copy(vals_hbm.at[pl.ds(0, CH)], bufa, sema).wait()
            pltpu.sync_copy(bufa, acc.at[dstv.at[c0]], add=True)
            gather(c0 + 2, bufa, sema)
            pltpu.make_async_copy(vals_hbm.at[pl.ds(0, CH)], bufb, semb).wait()
            pltpu.sync_copy(bufb, acc.at[dstv.at[c0 + 1]], add=True)
            return carry

        lax.fori_loop(0, (NCH - 1) // 2, pair, 0)
        pltpu.make_async_copy(vals_hbm.at[pl.ds(0, CH)], bufa, sema).wait()
        pltpu.sync_copy(bufa, acc.at[dstv.at[NCH - 1]], add=True)
        plsc.subcore_barrier()
        pltpu.sync_copy(acc.at[pl.ds(sid * ROWS_PER_SUB, ROWS_PER_SUB)],
                        out_hbm.at[cid, pl.ds(sid * ROWS_PER_SUB, ROWS_PER_SUB)])

        @pl.when(sid == NS - 1)
        def _out_tail():
            pltpu.sync_copy(acc.at[pl.ds(ROWS_TAIL_OFF, ROWS_TAIL)],
                            out_hbm.at[cid, pl.ds(ROWS_TAIL_OFF, ROWS_TAIL)])

    return seg_kernel(vals, src, dst3, zeros)


BT = 400


def _bspline_bases(z):
    nt = GRID_SIZE + 2 * SPLINE_ORDER + 1
    t = [0.5 * i - 2.5 for i in range(nt)]
    d = [z - ti for ti in t]
    s = [(z >= ti).astype(z.dtype) for ti in t]
    b = [s[i] - s[i + 1] for i in range(nt - 1)]
    for j in range(1, SPLINE_ORDER + 1):
        inv = 1.0 / (0.5 * j)
        b = [(d[i] * b[i] - d[i + j + 1] * b[i + 1]) * inv
             for i in range(len(b) - 1)]
    return b


def _silu(z):
    return z * (1.0 / (1.0 + jnp.exp(-z)))


def _kan1_body(x_ref, p_ref, bwt_ref, sw_ref, z_ref, h_ref):
    z = x_ref[...] + p_ref[0] + p_ref[1]
    z_ref[...] = z
    acc = jnp.dot(_silu(z), bwt_ref[...], preferred_element_type=jnp.float32)
    for c, bc in enumerate(_bspline_bases(z)):
        acc += jnp.dot(bc, sw_ref[c], preferred_element_type=jnp.float32)
    h_ref[...] = acc


def _kan1(x, p, bwt, sw):
    grid = (N_NODES // BT,)
    return pl.pallas_call(
        _kan1_body,
        grid=grid,
        in_specs=[
            pl.BlockSpec((BT, F), lambda i: (i, 0)),
            pl.BlockSpec((NC, BT, F), lambda i: (0, i, 0)),
            pl.BlockSpec((F, HIDDEN), lambda i: (0, 0)),
            pl.BlockSpec((COEF, F, HIDDEN), lambda i: (0, 0, 0)),
        ],
        out_specs=[
            pl.BlockSpec((BT, F), lambda i: (i, 0)),
            pl.BlockSpec((BT, HIDDEN), lambda i: (i, 0)),
        ],
        out_shape=[
            jax.ShapeDtypeStruct((N_NODES, F), jnp.float32),
            jax.ShapeDtypeStruct((N_NODES, HIDDEN), jnp.float32),
        ],
    )(x, p, bwt, sw)


def _kan2_body(z1_ref, h1_ref, q_ref, bwta_ref, bwtb_ref, swa_ref, swb_ref,
               o_ref):
    z1 = z1_ref[...]
    h2 = h1_ref[...] + q_ref[0] + q_ref[1]
    acc = jnp.dot(_silu(z1), bwta_ref[...], preferred_element_type=jnp.float32)
    acc += jnp.dot(_silu(h2), bwtb_ref[...], preferred_element_type=jnp.float32)
    for c, bc in enumerate(_bspline_bases(z1)):
        acc += jnp.dot(bc, swa_ref[c], preferred_element_type=jnp.float32)
    for c, bc in enumerate(_bspline_bases(h2)):
        acc += jnp.dot(bc, swb_ref[c], preferred_element_type=jnp.float32)
    o_ref[...] = acc


def _kan2(z1, h1, q, bwta, bwtb, swa, swb):
    grid = (N_NODES // BT,)
    return pl.pallas_call(
        _kan2_body,
        grid=grid,
        in_specs=[
            pl.BlockSpec((BT, F), lambda i: (i, 0)),
            pl.BlockSpec((BT, HIDDEN), lambda i: (i, 0)),
            pl.BlockSpec((NC, BT, HIDDEN), lambda i: (0, i, 0)),
            pl.BlockSpec((F, NUM_CLASSES), lambda i: (0, 0)),
            pl.BlockSpec((HIDDEN, NUM_CLASSES), lambda i: (0, 0)),
            pl.BlockSpec((COEF, F, NUM_CLASSES), lambda i: (0, 0, 0)),
            pl.BlockSpec((COEF, HIDDEN, NUM_CLASSES), lambda i: (0, 0, 0)),
        ],
        out_specs=pl.BlockSpec((BT, NUM_CLASSES), lambda i: (i, 0)),
        out_shape=jax.ShapeDtypeStruct((N_NODES, NUM_CLASSES), jnp.float32),
    )(z1, h1, q, bwta, bwtb, swa, swb)


def kernel(x, edge_index, base_w1, spline_w1, scaler1,
           base_w2, spline_w2, scaler2):
    src = edge_index[0]
    dst3 = edge_index[1].reshape(NW, NCH, CH)
    zeros = jnp.zeros((N_NODES, F), jnp.float32)

    bwt1 = base_w1.T
    sw1 = (spline_w1 * scaler1[:, :, None]).transpose(2, 1, 0)
    bwt2a = base_w2[:, :F].T
    bwt2b = base_w2[:, F:].T
    sw2 = (spline_w2 * scaler2[:, :, None]).transpose(2, 1, 0)
    sw2a = sw2[:, :F, :]
    sw2b = sw2[:, F:, :]

    p = _sc_segsum(x, src, dst3, zeros)
    z1, h1 = _kan1(x, p, bwt1, sw1)
    q = _sc_segsum(h1, src, dst3, zeros)
    return _kan2(z1, h1, q, bwt2a, bwt2b, sw2a, sw2b)


# speedup vs baseline: 3.0999x; 1.0156x over previous
import functools

import jax
import jax.numpy as jnp
from jax import lax
from jax.experimental import pallas as pl
from jax.experimental.pallas import tpu as pltpu
from jax.experimental.pallas import tpu_sc as plsc

N_NODES = 10000
N_EDGES = 320000
F = 128
HIDDEN = 128
NUM_CLASSES = 64
COEF = 7
SPLINE_ORDER = 3
GRID_SIZE = 4

NC = 2
NS = 16
NW = NC * NS
EPW = N_EDGES // NW
CH = 80
NCH = EPW // CH
ROWS_PER_SUB = 624
ROWS_TAIL_OFF = NS * ROWS_PER_SUB
ROWS_TAIL = N_NODES - ROWS_TAIL_OFF


def _sc_segsum(vals, src, dst3, zeros):
    mesh = plsc.VectorSubcoreMesh(core_axis_name="c", subcore_axis_name="s")

    @functools.partial(
        pl.kernel,
        out_type=jax.ShapeDtypeStruct((NC, N_NODES, F), jnp.float32),
        mesh=mesh,
        scratch_types=[
            pltpu.VMEM((EPW,), jnp.int32),
            pltpu.VMEM((NCH, CH), jnp.int32),
            pltpu.VMEM((CH, F), jnp.float32),
            pltpu.VMEM((CH, F), jnp.float32),
            pltpu.VMEM_SHARED((N_NODES, F), jnp.float32),
            pltpu.SemaphoreType.DMA,
            pltpu.SemaphoreType.DMA,
            pltpu.SemaphoreType.DMA,
        ],
    )
    def seg_kernel(vals_hbm, src_hbm, dst_hbm, zeros_hbm, out_hbm,
                   srcv, dstv, bufa, bufb, acc, sema, semb, semi):
        cid = lax.axis_index("c")
        sid = lax.axis_index("s")
        wid = sid * NC + cid
        base = wid * EPW
        cp_src = pltpu.async_copy(src_hbm.at[pl.ds(base, EPW)], srcv, semi)
        pltpu.sync_copy(dst_hbm.at[wid], dstv)
        pltpu.sync_copy(zeros_hbm.at[pl.ds(sid * ROWS_PER_SUB, ROWS_PER_SUB)],
                        acc.at[pl.ds(sid * ROWS_PER_SUB, ROWS_PER_SUB)])

        @pl.when(sid == NS - 1)
        def _zero_tail():
            pltpu.sync_copy(zeros_hbm.at[pl.ds(ROWS_TAIL_OFF, ROWS_TAIL)],
                            acc.at[pl.ds(ROWS_TAIL_OFF, ROWS_TAIL)])

        cp_src.wait()
        plsc.subcore_barrier()

        def gather(c, buf, sem):
            return pltpu.async_copy(vals_hbm.at[srcv.at[pl.ds(c * CH, CH)]],
                                    buf, sem)

        gather(0, bufa, sema)

        def pair(p, carry):
            c0 = 2 * p
            gather(c0 + 1, bufb, semb)
            pltpu.make_async_copy(vals_hbm.at[pl.ds(0, CH)], bufa, sema).wait()
            pltpu.sync_copy(bufa, acc.at[dstv.at[c0]], add=True)
            gather(c0 + 2, bufa, sema)
            pltpu.make_async_copy(vals_hbm.at[pl.ds(0, CH)], bufb, semb).wait()
            pltpu.sync_copy(bufb, acc.at[dstv.at[c0 + 1]], add=True)
            return carry

        lax.fori_loop(0, (NCH - 1) // 2, pair, 0)
        pltpu.make_async_copy(vals_hbm.at[pl.ds(0, CH)], bufa, sema).wait()
        pltpu.sync_copy(bufa, acc.at[dstv.at[NCH - 1]], add=True)
        plsc.subcore_barrier()
        pltpu.sync_copy(acc.at[pl.ds(sid * ROWS_PER_SUB, ROWS_PER_SUB)],
                        out_hbm.at[cid, pl.ds(sid * ROWS_PER_SUB, ROWS_PER_SUB)])

        @pl.when(sid == NS - 1)
        def _out_tail():
            pltpu.sync_copy(acc.at[pl.ds(ROWS_TAIL_OFF, ROWS_TAIL)],
                            out_hbm.at[cid, pl.ds(ROWS_TAIL_OFF, ROWS_TAIL)])

    return seg_kernel(vals, src, dst3, zeros)


BT = 1000


def _bspline_bases(z):
    nt = GRID_SIZE + 2 * SPLINE_ORDER + 1
    t = [0.5 * i - 2.5 for i in range(nt)]
    d = [z - ti for ti in t]
    s = [(z >= ti).astype(z.dtype) for ti in t]
    b = [s[i] - s[i + 1] for i in range(nt - 1)]
    for j in range(1, SPLINE_ORDER + 1):
        inv = 1.0 / (0.5 * j)
        b = [(d[i] * b[i] - d[i + j + 1] * b[i + 1]) * inv
             for i in range(len(b) - 1)]
    return b


def _silu(z):
    return z * (1.0 / (1.0 + jnp.exp(-z)))


def _kan1_body(x_ref, p_ref, bwt_ref, sw_ref, z_ref, h_ref):
    z = x_ref[...] + p_ref[0] + p_ref[1]
    z_ref[...] = z
    acc = jnp.dot(_silu(z), bwt_ref[...], preferred_element_type=jnp.float32)
    for c, bc in enumerate(_bspline_bases(z)):
        acc += jnp.dot(bc, sw_ref[c], preferred_element_type=jnp.float32)
    h_ref[...] = acc


def _kan1(x, p, bwt, sw):
    grid = (N_NODES // BT,)
    return pl.pallas_call(
        _kan1_body,
        grid=grid,
        in_specs=[
            pl.BlockSpec((BT, F), lambda i: (i, 0)),
            pl.BlockSpec((NC, BT, F), lambda i: (0, i, 0)),
            pl.BlockSpec((F, HIDDEN), lambda i: (0, 0)),
            pl.BlockSpec((COEF, F, HIDDEN), lambda i: (0, 0, 0)),
        ],
        out_specs=[
            pl.BlockSpec((BT, F), lambda i: (i, 0)),
            pl.BlockSpec((BT, HIDDEN), lambda i: (i, 0)),
        ],
        out_shape=[
            jax.ShapeDtypeStruct((N_NODES, F), jnp.float32),
            jax.ShapeDtypeStruct((N_NODES, HIDDEN), jnp.float32),
        ],
    )(x, p, bwt, sw)


def _kan2_body(z1_ref, h1_ref, q_ref, bwta_ref, bwtb_ref, swa_ref, swb_ref,
               o_ref):
    z1 = z1_ref[...]
    h2 = h1_ref[...] + q_ref[0] + q_ref[1]
    acc = jnp.dot(_silu(z1), bwta_ref[...], preferred_element_type=jnp.float32)
    acc += jnp.dot(_silu(h2), bwtb_ref[...], preferred_element_type=jnp.float32)
    for c, bc in enumerate(_bspline_bases(z1)):
        acc += jnp.dot(bc, swa_ref[c], preferred_element_type=jnp.float32)
    for c, bc in enumerate(_bspline_bases(h2)):
        acc += jnp.dot(bc, swb_ref[c], preferred_element_type=jnp.float32)
    o_ref[...] = acc


def _kan2(z1, h1, q, bwta, bwtb, swa, swb):
    grid = (N_NODES // BT,)
    return pl.pallas_call(
        _kan2_body,
        grid=grid,
        in_specs=[
            pl.BlockSpec((BT, F), lambda i: (i, 0)),
            pl.BlockSpec((BT, HIDDEN), lambda i: (i, 0)),
            pl.BlockSpec((NC, BT, HIDDEN), lambda i: (0, i, 0)),
            pl.BlockSpec((F, NUM_CLASSES), lambda i: (0, 0)),
            pl.BlockSpec((HIDDEN, NUM_CLASSES), lambda i: (0, 0)),
            pl.BlockSpec((COEF, F, NUM_CLASSES), lambda i: (0, 0, 0)),
            pl.BlockSpec((COEF, HIDDEN, NUM_CLASSES), lambda i: (0, 0, 0)),
        ],
        out_specs=pl.BlockSpec((BT, NUM_CLASSES), lambda i: (i, 0)),
        out_shape=jax.ShapeDtypeStruct((N_NODES, NUM_CLASSES), jnp.float32),
    )(z1, h1, q, bwta, bwtb, swa, swb)


def kernel(x, edge_index, base_w1, spline_w1, scaler1,
           base_w2, spline_w2, scaler2):
    src = edge_index[0]
    dst3 = edge_index[1].reshape(NW, NCH, CH)
    zeros = jnp.zeros((N_NODES, F), jnp.float32)

    bwt1 = base_w1.T
    sw1 = (spline_w1 * scaler1[:, :, None]).transpose(2, 1, 0)
    bwt2a = base_w2[:, :F].T
    bwt2b = base_w2[:, F:].T
    sw2 = (spline_w2 * scaler2[:, :, None]).transpose(2, 1, 0)
    sw2a = sw2[:, :F, :]
    sw2b = sw2[:, F:, :]

    p = _sc_segsum(x, src, dst3, zeros)
    z1, h1 = _kan1(x, p, bwt1, sw1)
    q = _sc_segsum(h1, src, dst3, zeros)
    return _kan2(z1, h1, q, bwt2a, bwt2b, sw2a, sw2b)


# speedup vs baseline: 3.1045x; 1.0015x over previous
import functools

import jax
import jax.numpy as jnp
from jax import lax
from jax.experimental import pallas as pl
from jax.experimental.pallas import tpu as pltpu
from jax.experimental.pallas import tpu_sc as plsc

N_NODES = 10000
N_EDGES = 320000
F = 128
HIDDEN = 128
NUM_CLASSES = 64
COEF = 7
SPLINE_ORDER = 3
GRID_SIZE = 4

NC = 2
NS = 16
NW = NC * NS
EPW = N_EDGES // NW
CH = 80
NCH = EPW // CH
ROWS_PER_SUB = 624
ROWS_TAIL_OFF = NS * ROWS_PER_SUB
ROWS_TAIL = N_NODES - ROWS_TAIL_OFF


def _sc_segsum(vals, src, dst3, zeros):
    mesh = plsc.VectorSubcoreMesh(core_axis_name="c", subcore_axis_name="s")

    @functools.partial(
        pl.kernel,
        out_type=jax.ShapeDtypeStruct((NC, N_NODES, F), jnp.float32),
        mesh=mesh,
        scratch_types=[
            pltpu.VMEM((EPW,), jnp.int32),
            pltpu.VMEM((NCH, CH), jnp.int32),
            pltpu.VMEM((CH, F), jnp.float32),
            pltpu.VMEM((CH, F), jnp.float32),
            pltpu.VMEM_SHARED((N_NODES, F), jnp.float32),
            pltpu.SemaphoreType.DMA,
            pltpu.SemaphoreType.DMA,
            pltpu.SemaphoreType.DMA,
        ],
    )
    def seg_kernel(vals_hbm, src_hbm, dst_hbm, zeros_hbm, out_hbm,
                   srcv, dstv, bufa, bufb, acc, sema, semb, semi):
        cid = lax.axis_index("c")
        sid = lax.axis_index("s")
        wid = sid * NC + cid
        base = wid * EPW
        cp_src = pltpu.async_copy(src_hbm.at[pl.ds(base, EPW)], srcv, semi)
        pltpu.sync_copy(dst_hbm.at[wid], dstv)
        pltpu.sync_copy(zeros_hbm.at[pl.ds(sid * ROWS_PER_SUB, ROWS_PER_SUB)],
                        acc.at[pl.ds(sid * ROWS_PER_SUB, ROWS_PER_SUB)])

        @pl.when(sid == NS - 1)
        def _zero_tail():
            pltpu.sync_copy(zeros_hbm.at[pl.ds(ROWS_TAIL_OFF, ROWS_TAIL)],
                            acc.at[pl.ds(ROWS_TAIL_OFF, ROWS_TAIL)])

        cp_src.wait()
        plsc.subcore_barrier()

        def gather(c, buf, sem):
            return pltpu.async_copy(vals_hbm.at[srcv.at[pl.ds(c * CH, CH)]],
                                    buf, sem)

        gather(0, bufa, sema)

        def pair(p, carry):
            c0 = 2 * p
            gather(c0 + 1, bufb, semb)
            pltpu.make_async_copy(vals_hbm.at[pl.ds(0, CH)], bufa, sema).wait()
            pltpu.sync_copy(bufa, acc.at[dstv.at[c0]], add=True)
            gather(c0 + 2, bufa, sema)
            pltpu.make_async_copy(vals_hbm.at[pl.ds(0, CH)], bufb, semb).wait()
            pltpu.sync_copy(bufb, acc.at[dstv.at[c0 + 1]], add=True)
            return carry

        lax.fori_loop(0, (NCH - 1) // 2, pair, 0)
        pltpu.make_async_copy(vals_hbm.at[pl.ds(0, CH)], bufa, sema).wait()
        pltpu.sync_copy(bufa, acc.at[dstv.at[NCH - 1]], add=True)
        plsc.subcore_barrier()
        pltpu.sync_copy(acc.at[pl.ds(sid * ROWS_PER_SUB, ROWS_PER_SUB)],
                        out_hbm.at[cid, pl.ds(sid * ROWS_PER_SUB, ROWS_PER_SUB)])

        @pl.when(sid == NS - 1)
        def _out_tail():
            pltpu.sync_copy(acc.at[pl.ds(ROWS_TAIL_OFF, ROWS_TAIL)],
                            out_hbm.at[cid, pl.ds(ROWS_TAIL_OFF, ROWS_TAIL)])

    return seg_kernel(vals, src, dst3, zeros)


BT = 2000


def _bspline_bases(z):
    nt = GRID_SIZE + 2 * SPLINE_ORDER + 1
    t = [0.5 * i - 2.5 for i in range(nt)]
    d = [z - ti for ti in t]
    s = [(z >= ti).astype(z.dtype) for ti in t]
    b = [s[i] - s[i + 1] for i in range(nt - 1)]
    for j in range(1, SPLINE_ORDER + 1):
        inv = 1.0 / (0.5 * j)
        b = [(d[i] * b[i] - d[i + j + 1] * b[i + 1]) * inv
             for i in range(len(b) - 1)]
    return b


def _silu(z):
    return z * (1.0 / (1.0 + jnp.exp(-z)))


def _kan1_body(x_ref, p_ref, bwt_ref, sw_ref, z_ref, h_ref):
    z = x_ref[...] + p_ref[0] + p_ref[1]
    z_ref[...] = z
    acc = jnp.dot(_silu(z), bwt_ref[...], preferred_element_type=jnp.float32)
    for c, bc in enumerate(_bspline_bases(z)):
        acc += jnp.dot(bc, sw_ref[c], preferred_element_type=jnp.float32)
    h_ref[...] = acc


def _kan1(x, p, bwt, sw):
    grid = (N_NODES // BT,)
    return pl.pallas_call(
        _kan1_body,
        grid=grid,
        in_specs=[
            pl.BlockSpec((BT, F), lambda i: (i, 0)),
            pl.BlockSpec((NC, BT, F), lambda i: (0, i, 0)),
            pl.BlockSpec((F, HIDDEN), lambda i: (0, 0)),
            pl.BlockSpec((COEF, F, HIDDEN), lambda i: (0, 0, 0)),
        ],
        out_specs=[
            pl.BlockSpec((BT, F), lambda i: (i, 0)),
            pl.BlockSpec((BT, HIDDEN), lambda i: (i, 0)),
        ],
        out_shape=[
            jax.ShapeDtypeStruct((N_NODES, F), jnp.float32),
            jax.ShapeDtypeStruct((N_NODES, HIDDEN), jnp.float32),
        ],
    )(x, p, bwt, sw)


def _kan2_body(z1_ref, h1_ref, q_ref, bwta_ref, bwtb_ref, swa_ref, swb_ref,
               o_ref):
    z1 = z1_ref[...]
    h2 = h1_ref[...] + q_ref[0] + q_ref[1]
    acc = jnp.dot(_silu(z1), bwta_ref[...], preferred_element_type=jnp.float32)
    acc += jnp.dot(_silu(h2), bwtb_ref[...], preferred_element_type=jnp.float32)
    for c, bc in enumerate(_bspline_bases(z1)):
        acc += jnp.dot(bc, swa_ref[c], preferred_element_type=jnp.float32)
    for c, bc in enumerate(_bspline_bases(h2)):
        acc += jnp.dot(bc, swb_ref[c], preferred_element_type=jnp.float32)
    o_ref[...] = acc


def _kan2(z1, h1, q, bwta, bwtb, swa, swb):
    grid = (N_NODES // BT,)
    return pl.pallas_call(
        _kan2_body,
        grid=grid,
        in_specs=[
            pl.BlockSpec((BT, F), lambda i: (i, 0)),
            pl.BlockSpec((BT, HIDDEN), lambda i: (i, 0)),
            pl.BlockSpec((NC, BT, HIDDEN), lambda i: (0, i, 0)),
            pl.BlockSpec((F, NUM_CLASSES), lambda i: (0, 0)),
            pl.BlockSpec((HIDDEN, NUM_CLASSES), lambda i: (0, 0)),
            pl.BlockSpec((COEF, F, NUM_CLASSES), lambda i: (0, 0, 0)),
            pl.BlockSpec((COEF, HIDDEN, NUM_CLASSES), lambda i: (0, 0, 0)),
        ],
        out_specs=pl.BlockSpec((BT, NUM_CLASSES), lambda i: (i, 0)),
        out_shape=jax.ShapeDtypeStruct((N_NODES, NUM_CLASSES), jnp.float32),
    )(z1, h1, q, bwta, bwtb, swa, swb)


def kernel(x, edge_index, base_w1, spline_w1, scaler1,
           base_w2, spline_w2, scaler2):
    src = edge_index[0]
    dst3 = edge_index[1].reshape(NW, NCH, CH)
    zeros = jnp.zeros((N_NODES, F), jnp.float32)

    bwt1 = base_w1.T
    sw1 = (spline_w1 * scaler1[:, :, None]).transpose(2, 1, 0)
    bwt2a = base_w2[:, :F].T
    bwt2b = base_w2[:, F:].T
    sw2 = (spline_w2 * scaler2[:, :, None]).transpose(2, 1, 0)
    sw2a = sw2[:, :F, :]
    sw2b = sw2[:, F:, :]

    p = _sc_segsum(x, src, dst3, zeros)
    z1, h1 = _kan1(x, p, bwt1, sw1)
    q = _sc_segsum(h1, src, dst3, zeros)
    return _kan2(z1, h1, q, bwt2a, bwt2b, sw2a, sw2b)
